# baseline (speedup 1.0000x reference)
import functools

import jax
import jax.numpy as jnp
from jax import lax
from jax.experimental import pallas as pl
from jax.experimental.pallas import tpu as pltpu
from jax.experimental.pallas import tpu_sc as plsc

_B = 16384
_NUMERIC_DIM = 16
_EMBED_DIM = 32
_OUTPUT_DIM = 128

_NC = 2
_NS = 16
_NW = _NC * _NS

_S = 512
_CAP = 256
_L = 16
_DUMP = _B

_VOCABS = {"host": 1000000, "dom": 100000, "cat": 1000}


def _shard(v):
    per_w = -(-v // _NW)
    return 128 * (-(-per_w // 128))


def _gather_table(tab_hbm, idx_hbm, out_hbm, v,
                  wid, idx_v, cidx_v, cpos_v, cposp_v, seg_bufs, row_v,
                  sems):
    shard = _shard(v)
    nseg = -(-shard // _S)
    if nseg % 2:
        nseg += 1
    lastbase = ((v - _S) // 128) * 128
    tail_base = v - (v % 128)
    tail_w = v % 128

    lo = wid * shard
    hi = lo + shard

    pltpu.sync_copy(idx_hbm, idx_v)

    def initb(j, _):
        cpos_v[j >> 4, pl.ds((j & 15) * _L, _L)] = jnp.full(
            (_L,), _DUMP, jnp.int32)
        return 0
    lax.fori_loop(0, (64 * _CAP) // _L, initb, 0)

    def compact(k, cnt):
        iv = idx_v[pl.ds(k * _L, _L)]
        m = (iv >= lo) & (iv < hi)
        dst = cnt + plsc.cumsum(m.astype(jnp.int32)) - 1
        pos = k * _L + lax.iota(jnp.int32, _L)
        plsc.store_scatter(cidx_v, [dst], iv, mask=m)
        plsc.store_scatter(cpos_v, [dst >> 8, dst & (_CAP - 1)], pos, mask=m)
        return cnt + jnp.sum(m.astype(jnp.int32), axis=0)
    cnt = lax.fori_loop(0, _B // _L, compact, jnp.int32(0))

    npass = (cnt + (_CAP - 1)) // _CAP

    def scan_window(p, base, buf, wlim):
        def vreg(r, _):
            g = p * _CAP + r * _L
            iv = cidx_v[pl.ds(g, _L)]
            inlist = (r * _L + lax.iota(jnp.int32, _L)) < (cnt - p * _CAP)
            m = inlist & (iv >= base) & (iv < base + wlim)

            @pl.when(jnp.any(m))
            def _():
                lv = iv - base
                rows = r * _L + lax.iota(jnp.int32, _L)
                for c in range(_EMBED_DIM):
                    cc = jnp.full((_L,), c, jnp.int32)
                    vals = plsc.load_gather(buf, [cc, lv], mask=m)
                    plsc.store_scatter(row_v, [rows, cc], vals, mask=m)
            return 0
        lax.fori_loop(0, _CAP // _L, vreg, 0)

    del tail_base, tail_w

    def one_pass(p, _):
        def segpair(sh, _):
            for par in range(2):
                s = sh * 2 + par
                base = jnp.minimum(lo + s * _S, lastbase)
                base = pl.multiple_of(base, 128)
                buf = seg_bufs[par]
                cp = pltpu.async_copy(
                    tab_hbm.at[:, pl.ds(base, _S)], buf, sems[par])
                cp.wait()
                scan_window(p, base, buf, _S)
            return 0
        lax.fori_loop(0, nseg // 2, segpair, 0)

        def poscopy(j, _):
            cposp_v[pl.ds(j * _L, _L)] = cpos_v[p, pl.ds(j * _L, _L)]
            return 0
        lax.fori_loop(0, _CAP // _L, poscopy, 0)
        pltpu.async_copy(row_v, out_hbm.at[cposp_v], sems[0]).wait()
        return 0

    lax.fori_loop(0, npass, one_pass, 0)


@functools.cache
def _make_sc_gather3():
    mesh = plsc.VectorSubcoreMesh(core_axis_name="c", subcore_axis_name="s")

    @functools.partial(
        pl.kernel,
        out_type=tuple(
            jax.ShapeDtypeStruct((_B + 8, _OUTPUT_DIM), jnp.float32)
            for _ in range(3)),
        mesh=mesh,
        scratch_types=[
            pltpu.VMEM((_B,), jnp.int32),
            pltpu.VMEM((_B + _L,), jnp.int32),
            pltpu.VMEM((64, _CAP), jnp.int32),
            pltpu.VMEM((_CAP,), jnp.int32),
            pltpu.VMEM((_EMBED_DIM, _S), jnp.float32),
            pltpu.VMEM((_EMBED_DIM, _S), jnp.float32),
            pltpu.VMEM((_CAP, _OUTPUT_DIM), jnp.float32),
            pltpu.SemaphoreType.DMA,
            pltpu.SemaphoreType.DMA,
        ],
        compiler_params=pltpu.CompilerParams(needs_layout_passes=False),
    )
    def _sc_gather3(cat_idx_hbm, host_idx_hbm, dom_idx_hbm,
                    cat_t_hbm, host_t_hbm, dom_t_hbm,
                    cat_out, host_out, dom_out,
                    idx_v, cidx_v, cpos_v, cposp_v, seg0_v, seg1_v, row_v,
                    s0, s1):
        wid = lax.axis_index("s") * _NC + lax.axis_index("c")
        for idx_hbm, tab_hbm, out_hbm, v in (
                (host_idx_hbm, host_t_hbm, host_out, _VOCABS["host"]),
                (dom_idx_hbm, dom_t_hbm, dom_out, _VOCABS["dom"]),
                (cat_idx_hbm, cat_t_hbm, cat_out, _VOCABS["cat"]),
        ):
            _gather_table(tab_hbm, idx_hbm, out_hbm, v, wid,
                          idx_v, cidx_v, cpos_v, cposp_v,
                          (seg0_v, seg1_v), row_v, (s0, s1))

    return _sc_gather3


_ROWS = 2048


def _dot0(a, w):
    return lax.dot_general(a, w, (((0,), (0,)), ((), ())),
                           preferred_element_type=jnp.float32)


def _emb_h(g_ref, idx_ref, m_ref, w_slice, v):
    tail_base = v - (v % 128)
    idx = idx_ref[...]
    tm = idx >= tail_base
    emb = jnp.where(tm, 0.0, g_ref[...][:, :_EMBED_DIM])
    h = jnp.dot(emb, w_slice, preferred_element_type=jnp.float32)
    if v % 128:
        rows = idx.shape[0]
        lane = lax.broadcasted_iota(jnp.int32, (rows, _OUTPUT_DIM), 1)
        oh = ((idx - tail_base) == lane) & tm
        h = h + jnp.dot(oh.astype(jnp.float32), m_ref[...],
                        preferred_element_type=jnp.float32)
    return h


def _tc_body(num_ref, ci_ref, hi_ref, di_ref, cat_ref, host_ref, dom_ref,
             w_ref, mc_ref, mh_ref, md_ref, b_ref, g_ref, be_ref, out_ref):
    w = w_ref[...]
    h = (_dot0(num_ref[...], w[0:16])
         + _emb_h(cat_ref, ci_ref, mc_ref, w[16:48], _VOCABS["cat"])
         + _emb_h(host_ref, hi_ref, mh_ref, w[48:80], _VOCABS["host"])
         + _emb_h(dom_ref, di_ref, md_ref, w[80:112], _VOCABS["dom"])
         + b_ref[...])
    mean = jnp.mean(h, axis=-1, keepdims=True)
    var = jnp.mean(jnp.square(h - mean), axis=-1, keepdims=True)
    y = (h - mean) * lax.rsqrt(var + 1e-5) * g_ref[...] + be_ref[...]
    out_ref[...] = y * 0.5 * (1.0 + lax.erf(y * 0.7071067811865476))


def _tc_dense(num_t, ci2, hi2, di2, cat_g, host_g, dom_g,
              W, mc, mh, md, b, gamma, beta):
    grid = _B // _ROWS
    full = lambda i: (0, 0)
    return pl.pallas_call(
        _tc_body,
        grid=(grid,),
        in_specs=[
            pl.BlockSpec((_NUMERIC_DIM, _ROWS), lambda i: (0, i)),
            pl.BlockSpec((_ROWS, 1), lambda i: (i, 0)),
            pl.BlockSpec((_ROWS, 1), lambda i: (i, 0)),
            pl.BlockSpec((_ROWS, 1), lambda i: (i, 0)),
            pl.BlockSpec((_ROWS, _OUTPUT_DIM), lambda i: (i, 0)),
            pl.BlockSpec((_ROWS, _OUTPUT_DIM), lambda i: (i, 0)),
            pl.BlockSpec((_ROWS, _OUTPUT_DIM), lambda i: (i, 0)),
            pl.BlockSpec((_NUMERIC_DIM + 3 * _EMBED_DIM, _OUTPUT_DIM), full),
            pl.BlockSpec((_OUTPUT_DIM, _OUTPUT_DIM), full),
            pl.BlockSpec((_OUTPUT_DIM, _OUTPUT_DIM), full),
            pl.BlockSpec((_OUTPUT_DIM, _OUTPUT_DIM), full),
            pl.BlockSpec((1, _OUTPUT_DIM), full),
            pl.BlockSpec((1, _OUTPUT_DIM), full),
            pl.BlockSpec((1, _OUTPUT_DIM), full),
        ],
        out_specs=pl.BlockSpec((_ROWS, _OUTPUT_DIM), lambda i: (i, 0)),
        out_shape=jax.ShapeDtypeStruct((_B, _OUTPUT_DIM), jnp.float32),
    )(num_t, ci2, hi2, di2, cat_g, host_g, dom_g, W, mc, mh, md,
      b.reshape(1, _OUTPUT_DIM), gamma.reshape(1, _OUTPUT_DIM),
      beta.reshape(1, _OUTPUT_DIM))


def _tail_matrix(table, w_slice):
    v = table.shape[0]
    tw = v % 128
    tail = table[v - tw:]
    return jnp.pad(tail, ((0, 128 - tw), (0, 0))) @ w_slice


def kernel(meta_numeric, meta_category_id, meta_host_id, meta_domain_id,
           cat_table, host_table, domain_table, W, b, gamma, beta):
    ci = meta_category_id.astype(jnp.int32)
    hi = meta_host_id.astype(jnp.int32)
    di = meta_domain_id.astype(jnp.int32)
    cat_g, host_g, dom_g = _make_sc_gather3()(
        ci, hi, di, cat_table.T, host_table.T, domain_table.T)
    mc = _tail_matrix(cat_table, W[16:48])
    mh = _tail_matrix(host_table, W[48:80])
    md = _tail_matrix(domain_table, W[80:112])
    return _tc_dense(meta_numeric.T, ci.reshape(-1, 1), hi.reshape(-1, 1),
                     di.reshape(-1, 1), cat_g, host_g, dom_g,
                     W, mc, mh, md, b, gamma, beta)


# speedup vs baseline: 1.1268x; 1.1268x over previous
import functools

import jax
import jax.numpy as jnp
from jax import lax
from jax.experimental import pallas as pl
from jax.experimental.pallas import tpu as pltpu
from jax.experimental.pallas import tpu_sc as plsc

_B = 16384
_NUMERIC_DIM = 16
_EMBED_DIM = 32
_OUTPUT_DIM = 128

_NC = 2
_NS = 16
_NW = _NC * _NS

_S = 256
_LCAP = 640
_SCH = 64
_L = 16
_DUMP = _B

_VOCABS = {"host": 1000000, "dom": 100000, "cat": 1000}


def _shard(v):
    per_w = -(-v // _NW)
    return 128 * (-(-per_w // 128))


def _gather_table(tab_hbm, idx_hbm, out_hbm, v, wid, idx_v,
                  cposp_v, seg_bufs, row32_v, rb_v, sems, ssem):
    shard = _shard(v)
    nseg = -(-shard // _S)
    if nseg % 2:
        nseg += 1
    lastbase = ((v - _S) // 128) * 128
    lo = wid * shard
    hi = lo + shard

    pltpu.sync_copy(idx_hbm, idx_v)

    def compact(k, cnt):
        iv = idx_v[pl.ds(k * _L, _L)]
        m = (iv >= lo) & (iv < hi)
        dst = cnt + plsc.cumsum(m.astype(jnp.int32)) - 1
        pos = k * _L + lax.iota(jnp.int32, _L)
        entry = ((iv - lo) << 14) | pos
        plsc.store_scatter(idx_v, [dst], entry, mask=m)
        return cnt + jnp.sum(m.astype(jnp.int32), axis=0)
    cnt = lax.fori_loop(0, _B // _L, compact, jnp.int32(0))

    npass = (cnt + (_LCAP - 1)) // _LCAP

    def wbase(s):
        return pl.multiple_of(jnp.minimum(lo + s * _S, lastbase), 128)

    def stage(s, par):
        return pltpu.async_copy(
            tab_hbm.at[:, pl.ds(wbase(s), _S)], seg_bufs[par], sems[par])

    def scan_window(p, base, buf):
        lbase = base - lo

        def vreg(r, _):
            g = p * _LCAP + r * _L
            ev = idx_v[pl.ds(g, _L)]
            liv = ev >> 14
            inlist = (r * _L + lax.iota(jnp.int32, _L)) < (cnt - p * _LCAP)
            m = inlist & (liv >= lbase) & (liv < lbase + _S)

            @pl.when(jnp.any(m))
            def _():
                lv = liv - lbase
                rows = r * _L + lax.iota(jnp.int32, _L)
                for c in range(_EMBED_DIM):
                    cc = jnp.full((_L,), c, jnp.int32)
                    vals = plsc.load_gather(buf, [cc, lv], mask=m)
                    plsc.store_scatter(row32_v, [rows, cc], vals, mask=m)
            return 0
        lax.fori_loop(0, _LCAP // _L, vreg, 0)

    def one_pass(p, _):
        stage(0, 0)
        stage(1, 1)

        def segpair(sh, _):
            for par in range(2):
                s = sh * 2 + par
                base = wbase(s)
                pltpu.make_async_copy(
                    tab_hbm.at[:, pl.ds(base, _S)], seg_bufs[par],
                    sems[par]).wait()
                scan_window(p, base, seg_bufs[par])

                @pl.when(s + 2 < nseg)
                def _():
                    stage(s + 2, par)
            return 0
        lax.fori_loop(0, nseg // 2, segpair, 0)

        handle = None
        for q in range(_LCAP // _SCH):
            if handle is not None:
                handle.wait()

            def expand(rr, _, _q=q):
                src = _q * _SCH + rr
                rb_v[rr, pl.ds(0, _L)] = row32_v[src, pl.ds(0, _L)]
                rb_v[rr, pl.ds(_L, _L)] = row32_v[src, pl.ds(_L, _L)]
                return 0
            lax.fori_loop(0, _SCH, expand, 0)

            def poscopy(j, _, _q=q):
                g = p * _LCAP + _q * _SCH + j * _L
                ev = idx_v[pl.ds(g, _L)]
                valid = (g + lax.iota(jnp.int32, _L)) < cnt
                cposp_v[pl.ds(j * _L, _L)] = jnp.where(
                    valid, ev & ((1 << 14) - 1), _DUMP)
                return 0
            lax.fori_loop(0, _SCH // _L, poscopy, 0)
            handle = pltpu.async_copy(rb_v, out_hbm.at[cposp_v], ssem)
        handle.wait()
        return 0

    lax.fori_loop(0, npass, one_pass, 0)


@functools.cache
def _make_sc_gather3():
    mesh = plsc.VectorSubcoreMesh(core_axis_name="c", subcore_axis_name="s")

    @functools.partial(
        pl.kernel,
        out_type=tuple(
            jax.ShapeDtypeStruct((_B + 8, _OUTPUT_DIM), jnp.float32)
            for _ in range(3)),
        mesh=mesh,
        scratch_types=[
            pltpu.VMEM((_B,), jnp.int32),
            pltpu.VMEM((_SCH,), jnp.int32),
            pltpu.VMEM((_EMBED_DIM, _S), jnp.float32),
            pltpu.VMEM((_EMBED_DIM, _S), jnp.float32),
            pltpu.VMEM((_LCAP, _EMBED_DIM), jnp.float32),
            pltpu.VMEM((_SCH, _OUTPUT_DIM), jnp.float32),
            pltpu.SemaphoreType.DMA,
            pltpu.SemaphoreType.DMA,
            pltpu.SemaphoreType.DMA,
        ],
        compiler_params=pltpu.CompilerParams(needs_layout_passes=False),
    )
    def _sc_gather3(cat_idx_hbm, host_idx_hbm, dom_idx_hbm,
                    cat_t_hbm, host_t_hbm, dom_t_hbm,
                    cat_out, host_out, dom_out,
                    idx_v, cposp_v, seg0_v, seg1_v, row32_v, rb_v,
                    s0, s1, s2):
        wid = lax.axis_index("s") * _NC + lax.axis_index("c")
        for idx_hbm, tab_hbm, out_hbm, v in (
                (host_idx_hbm, host_t_hbm, host_out, _VOCABS["host"]),
                (dom_idx_hbm, dom_t_hbm, dom_out, _VOCABS["dom"]),
                (cat_idx_hbm, cat_t_hbm, cat_out, _VOCABS["cat"]),
        ):
            _gather_table(tab_hbm, idx_hbm, out_hbm, v, wid,
                          idx_v, cposp_v, (seg0_v, seg1_v), row32_v, rb_v,
                          (s0, s1), s2)

    return _sc_gather3


_ROWS = 2048


def _dot0(a, w):
    return lax.dot_general(a, w, (((0,), (0,)), ((), ())),
                           preferred_element_type=jnp.float32)


def _emb_h(g_ref, idx_ref, m_ref, w_slice, v):
    tail_base = v - (v % 128)
    idx = idx_ref[...]
    tm = idx >= tail_base
    emb = jnp.where(tm, 0.0, g_ref[...][:, :_EMBED_DIM])
    h = jnp.dot(emb, w_slice, preferred_element_type=jnp.float32)
    if v % 128:
        rows = idx.shape[0]
        lane = lax.broadcasted_iota(jnp.int32, (rows, _OUTPUT_DIM), 1)
        oh = ((idx - tail_base) == lane) & tm
        h = h + jnp.dot(oh.astype(jnp.float32), m_ref[...],
                        preferred_element_type=jnp.float32)
    return h


def _tc_body(num_ref, ci_ref, hi_ref, di_ref, cat_ref, host_ref, dom_ref,
             w_ref, mc_ref, mh_ref, md_ref, b_ref, g_ref, be_ref, out_ref):
    w = w_ref[...]
    h = (_dot0(num_ref[...], w[0:16])
         + _emb_h(cat_ref, ci_ref, mc_ref, w[16:48], _VOCABS["cat"])
         + _emb_h(host_ref, hi_ref, mh_ref, w[48:80], _VOCABS["host"])
         + _emb_h(dom_ref, di_ref, md_ref, w[80:112], _VOCABS["dom"])
         + b_ref[...])
    mean = jnp.mean(h, axis=-1, keepdims=True)
    var = jnp.mean(jnp.square(h - mean), axis=-1, keepdims=True)
    y = (h - mean) * lax.rsqrt(var + 1e-5) * g_ref[...] + be_ref[...]
    out_ref[...] = y * 0.5 * (1.0 + lax.erf(y * 0.7071067811865476))


def _tc_dense(num_t, ci2, hi2, di2, cat_g, host_g, dom_g,
              W, mc, mh, md, b, gamma, beta):
    grid = _B // _ROWS
    full = lambda i: (0, 0)
    return pl.pallas_call(
        _tc_body,
        grid=(grid,),
        in_specs=[
            pl.BlockSpec((_NUMERIC_DIM, _ROWS), lambda i: (0, i)),
            pl.BlockSpec((_ROWS, 1), lambda i: (i, 0)),
            pl.BlockSpec((_ROWS, 1), lambda i: (i, 0)),
            pl.BlockSpec((_ROWS, 1), lambda i: (i, 0)),
            pl.BlockSpec((_ROWS, _OUTPUT_DIM), lambda i: (i, 0)),
            pl.BlockSpec((_ROWS, _OUTPUT_DIM), lambda i: (i, 0)),
            pl.BlockSpec((_ROWS, _OUTPUT_DIM), lambda i: (i, 0)),
            pl.BlockSpec((_NUMERIC_DIM + 3 * _EMBED_DIM, _OUTPUT_DIM), full),
            pl.BlockSpec((_OUTPUT_DIM, _OUTPUT_DIM), full),
            pl.BlockSpec((_OUTPUT_DIM, _OUTPUT_DIM), full),
            pl.BlockSpec((_OUTPUT_DIM, _OUTPUT_DIM), full),
            pl.BlockSpec((1, _OUTPUT_DIM), full),
            pl.BlockSpec((1, _OUTPUT_DIM), full),
            pl.BlockSpec((1, _OUTPUT_DIM), full),
        ],
        out_specs=pl.BlockSpec((_ROWS, _OUTPUT_DIM), lambda i: (i, 0)),
        out_shape=jax.ShapeDtypeStruct((_B, _OUTPUT_DIM), jnp.float32),
    )(num_t, ci2, hi2, di2, cat_g, host_g, dom_g, W, mc, mh, md,
      b.reshape(1, _OUTPUT_DIM), gamma.reshape(1, _OUTPUT_DIM),
      beta.reshape(1, _OUTPUT_DIM))


def _tail_matrix(table, w_slice):
    v = table.shape[0]
    tw = v % 128
    tail = table[v - tw:]
    return jnp.pad(tail, ((0, 128 - tw), (0, 0))) @ w_slice


def kernel(meta_numeric, meta_category_id, meta_host_id, meta_domain_id,
           cat_table, host_table, domain_table, W, b, gamma, beta):
    ci = meta_category_id.astype(jnp.int32)
    hi = meta_host_id.astype(jnp.int32)
    di = meta_domain_id.astype(jnp.int32)
    cat_g, host_g, dom_g = _make_sc_gather3()(
        ci, hi, di, cat_table.T, host_table.T, domain_table.T)
    mc = _tail_matrix(cat_table, W[16:48])
    mh = _tail_matrix(host_table, W[48:80])
    md = _tail_matrix(domain_table, W[80:112])
    return _tc_dense(meta_numeric.T, ci.reshape(-1, 1), hi.reshape(-1, 1),
                     di.reshape(-1, 1), cat_g, host_g, dom_g,
                     W, mc, mh, md, b, gamma, beta)


# speedup vs baseline: 1.6843x; 1.4948x over previous
import functools

import jax
import jax.numpy as jnp
from jax import lax
from jax.experimental import pallas as pl
from jax.experimental.pallas import tpu as pltpu
from jax.experimental.pallas import tpu_sc as plsc

_B = 16384
_NUMERIC_DIM = 16
_EMBED_DIM = 32
_OUTPUT_DIM = 128

_NC = 2
_NS = 16
_NW = _NC * _NS

_S = 384
_LCAP = 576
_SCH = 64
_L = 16
_DUMP = _B
_LB = (-(-_B // _LCAP)) * _LCAP

_VOCABS = {"host": 1000000, "dom": 100000, "cat": 1000}


def _shard(v):
    per_w = -(-v // _NW)
    return 128 * (-(-per_w // 128))


def _gather_table(tab_hbm, idx_hbm, out_hbm, v, wid, idx_v,
                  cposp_v, seg_bufs, row32_v, rb_v, sems, ssem):
    shard = _shard(v)
    nseg = -(-shard // _S)
    if nseg % 2:
        nseg += 1
    lastbase = ((v - _S) // 128) * 128
    lo = wid * shard
    hi = lo + shard

    pltpu.sync_copy(idx_hbm, idx_v.at[pl.ds(0, _B)])

    def compact(k, cnt):
        iv = idx_v[pl.ds(k * _L, _L)]
        m = (iv >= lo) & (iv < hi)
        dst = cnt + plsc.cumsum(m.astype(jnp.int32)) - 1
        pos = k * _L + lax.iota(jnp.int32, _L)
        entry = ((iv - lo) << 15) | pos
        plsc.store_scatter(idx_v, [dst], entry, mask=m)
        return cnt + jnp.sum(m.astype(jnp.int32), axis=0)
    cnt = lax.fori_loop(0, _B // _L, compact, jnp.int32(0))

    npass = (cnt + (_LCAP - 1)) // _LCAP

    sentinel = (0x7FFF << 15) | _DUMP

    def filltail(t, _):
        j = cnt + t * _L
        plsc.store_scatter(
            idx_v, [j + lax.iota(jnp.int32, _L)],
            jnp.full((_L,), sentinel, jnp.int32),
            mask=(j + lax.iota(jnp.int32, _L)) < _LB)
        return 0
    lax.fori_loop(0, (npass * _LCAP - cnt + _L - 1) // _L, filltail, 0)

    def wbase(s):
        return pl.multiple_of(jnp.minimum(lo + s * _S, lastbase), 128)

    def stage(s, par):
        return pltpu.async_copy(
            tab_hbm.at[:, pl.ds(wbase(s), _S)], seg_bufs[par], sems[par])

    def scan_window(p, base, buf):
        lbase = base - lo

        def vreg(r, _):
            g = p * _LCAP + r * _L
            ev = idx_v[pl.ds(g, _L)]
            liv = ev >> 15
            m = (liv >= lbase) & (liv < lbase + _S)

            @pl.when(jnp.any(m))
            def _():
                lv = liv - lbase
                rows = r * _L + lax.iota(jnp.int32, _L)
                for c in range(_EMBED_DIM):
                    cc = jnp.full((_L,), c, jnp.int32)
                    vals = plsc.load_gather(buf, [cc, lv], mask=m)
                    plsc.store_scatter(row32_v, [rows, cc], vals, mask=m)
            return 0
        lax.fori_loop(0, _LCAP // _L, vreg, 0)

    def one_pass(p, _):
        stage(0, 0)
        stage(1, 1)

        def segpair(sh, _):
            for par in range(2):
                s = sh * 2 + par
                base = wbase(s)
                pltpu.make_async_copy(
                    tab_hbm.at[:, pl.ds(base, _S)], seg_bufs[par],
                    sems[par]).wait()
                scan_window(p, base, seg_bufs[par])

                @pl.when(s + 2 < nseg)
                def _():
                    stage(s + 2, par)
            return 0
        lax.fori_loop(0, nseg // 2, segpair, 0)

        handle = None
        for q in range(_LCAP // _SCH):
            if handle is not None:
                handle.wait()

            def expand(rr, _, _q=q):
                src = _q * _SCH + rr
                rb_v[rr, pl.ds(0, _L)] = row32_v[src, pl.ds(0, _L)]
                rb_v[rr, pl.ds(_L, _L)] = row32_v[src, pl.ds(_L, _L)]
                return 0
            lax.fori_loop(0, _SCH, expand, 0)

            def poscopy(j, _, _q=q):
                g = p * _LCAP + _q * _SCH + j * _L
                ev = idx_v[pl.ds(g, _L)]
                cposp_v[pl.ds(j * _L, _L)] = ev & ((1 << 15) - 1)
                return 0
            lax.fori_loop(0, _SCH // _L, poscopy, 0)
            handle = pltpu.async_copy(rb_v, out_hbm.at[cposp_v], ssem)
        handle.wait()
        return 0

    lax.fori_loop(0, npass, one_pass, 0)


@functools.cache
def _make_sc_gather3():
    mesh = plsc.VectorSubcoreMesh(core_axis_name="c", subcore_axis_name="s")

    @functools.partial(
        pl.kernel,
        out_type=tuple(
            jax.ShapeDtypeStruct((_B + 8, _OUTPUT_DIM), jnp.float32)
            for _ in range(3)),
        mesh=mesh,
        scratch_types=[
            pltpu.VMEM((_LB,), jnp.int32),
            pltpu.VMEM((_SCH,), jnp.int32),
            pltpu.VMEM((_EMBED_DIM, _S), jnp.float32),
            pltpu.VMEM((_EMBED_DIM, _S), jnp.float32),
            pltpu.VMEM((_LCAP, _EMBED_DIM), jnp.float32),
            pltpu.VMEM((_SCH, _OUTPUT_DIM), jnp.float32),
            pltpu.SemaphoreType.DMA,
            pltpu.SemaphoreType.DMA,
            pltpu.SemaphoreType.DMA,
        ],
        compiler_params=pltpu.CompilerParams(needs_layout_passes=False),
    )
    def _sc_gather3(cat_idx_hbm, host_idx_hbm, dom_idx_hbm,
                    cat_t_hbm, host_t_hbm, dom_t_hbm,
                    cat_out, host_out, dom_out,
                    idx_v, cposp_v, seg0_v, seg1_v, row32_v, rb_v,
                    s0, s1, s2):
        wid = lax.axis_index("s") * _NC + lax.axis_index("c")
        for idx_hbm, tab_hbm, out_hbm, v in (
                (host_idx_hbm, host_t_hbm, host_out, _VOCABS["host"]),
                (dom_idx_hbm, dom_t_hbm, dom_out, _VOCABS["dom"]),
                (cat_idx_hbm, cat_t_hbm, cat_out, _VOCABS["cat"]),
        ):
            _gather_table(tab_hbm, idx_hbm, out_hbm, v, wid,
                          idx_v, cposp_v, (seg0_v, seg1_v), row32_v, rb_v,
                          (s0, s1), s2)

    return _sc_gather3


_ROWS = 2048


def _dot0(a, w):
    return lax.dot_general(a, w, (((0,), (0,)), ((), ())),
                           preferred_element_type=jnp.float32)


def _emb_h(g_ref, idx_ref, m_ref, w_slice, v):
    tail_base = v - (v % 128)
    idx = idx_ref[...]
    tm = idx >= tail_base
    emb = jnp.where(tm, 0.0, g_ref[...][:, :_EMBED_DIM])
    h = jnp.dot(emb, w_slice, preferred_element_type=jnp.float32)
    if v % 128:
        rows = idx.shape[0]
        lane = lax.broadcasted_iota(jnp.int32, (rows, _OUTPUT_DIM), 1)
        oh = ((idx - tail_base) == lane) & tm
        h = h + jnp.dot(oh.astype(jnp.float32), m_ref[...],
                        preferred_element_type=jnp.float32)
    return h


def _tc_body(num_ref, ci_ref, hi_ref, di_ref, cat_ref, host_ref, dom_ref,
             w_ref, mc_ref, mh_ref, md_ref, b_ref, g_ref, be_ref, out_ref):
    w = w_ref[...]
    h = (_dot0(num_ref[...], w[0:16])
         + _emb_h(cat_ref, ci_ref, mc_ref, w[16:48], _VOCABS["cat"])
         + _emb_h(host_ref, hi_ref, mh_ref, w[48:80], _VOCABS["host"])
         + _emb_h(dom_ref, di_ref, md_ref, w[80:112], _VOCABS["dom"])
         + b_ref[...])
    mean = jnp.mean(h, axis=-1, keepdims=True)
    var = jnp.mean(jnp.square(h - mean), axis=-1, keepdims=True)
    y = (h - mean) * lax.rsqrt(var + 1e-5) * g_ref[...] + be_ref[...]
    out_ref[...] = y * 0.5 * (1.0 + lax.erf(y * 0.7071067811865476))


def _tc_dense(num_t, ci2, hi2, di2, cat_g, host_g, dom_g,
              W, mc, mh, md, b, gamma, beta):
    grid = _B // _ROWS
    full = lambda i: (0, 0)
    return pl.pallas_call(
        _tc_body,
        grid=(grid,),
        in_specs=[
            pl.BlockSpec((_NUMERIC_DIM, _ROWS), lambda i: (0, i)),
            pl.BlockSpec((_ROWS, 1), lambda i: (i, 0)),
            pl.BlockSpec((_ROWS, 1), lambda i: (i, 0)),
            pl.BlockSpec((_ROWS, 1), lambda i: (i, 0)),
            pl.BlockSpec((_ROWS, _OUTPUT_DIM), lambda i: (i, 0)),
            pl.BlockSpec((_ROWS, _OUTPUT_DIM), lambda i: (i, 0)),
            pl.BlockSpec((_ROWS, _OUTPUT_DIM), lambda i: (i, 0)),
            pl.BlockSpec((_NUMERIC_DIM + 3 * _EMBED_DIM, _OUTPUT_DIM), full),
            pl.BlockSpec((_OUTPUT_DIM, _OUTPUT_DIM), full),
            pl.BlockSpec((_OUTPUT_DIM, _OUTPUT_DIM), full),
            pl.BlockSpec((_OUTPUT_DIM, _OUTPUT_DIM), full),
            pl.BlockSpec((1, _OUTPUT_DIM), full),
            pl.BlockSpec((1, _OUTPUT_DIM), full),
            pl.BlockSpec((1, _OUTPUT_DIM), full),
        ],
        out_specs=pl.BlockSpec((_ROWS, _OUTPUT_DIM), lambda i: (i, 0)),
        out_shape=jax.ShapeDtypeStruct((_B, _OUTPUT_DIM), jnp.float32),
    )(num_t, ci2, hi2, di2, cat_g, host_g, dom_g, W, mc, mh, md,
      b.reshape(1, _OUTPUT_DIM), gamma.reshape(1, _OUTPUT_DIM),
      beta.reshape(1, _OUTPUT_DIM))


def _tail_matrix(table, w_slice):
    v = table.shape[0]
    tw = v % 128
    tail = table[v - tw:]
    return jnp.pad(tail, ((0, 128 - tw), (0, 0))) @ w_slice


def kernel(meta_numeric, meta_category_id, meta_host_id, meta_domain_id,
           cat_table, host_table, domain_table, W, b, gamma, beta):
    ci = meta_category_id.astype(jnp.int32)
    hi = meta_host_id.astype(jnp.int32)
    di = meta_domain_id.astype(jnp.int32)
    cat_g, host_g, dom_g = _make_sc_gather3()(
        ci, hi, di, cat_table.T, host_table.T, domain_table.T)
    mc = _tail_matrix(cat_table, W[16:48])
    mh = _tail_matrix(host_table, W[48:80])
    md = _tail_matrix(domain_table, W[80:112])
    return _tc_dense(meta_numeric.T, ci.reshape(-1, 1), hi.reshape(-1, 1),
                     di.reshape(-1, 1), cat_g, host_g, dom_g,
                     W, mc, mh, md, b, gamma, beta)


# speedup vs baseline: 1.6878x; 1.0021x over previous
import functools

import jax
import jax.numpy as jnp
from jax import lax
from jax.experimental import pallas as pl
from jax.experimental.pallas import tpu as pltpu
from jax.experimental.pallas import tpu_sc as plsc

_B = 16384
_NUMERIC_DIM = 16
_EMBED_DIM = 32
_OUTPUT_DIM = 128

_NC = 2
_NS = 16
_NW = _NC * _NS

_S = 384
_LCAP = 576
_SCH = 64
_L = 16
_DUMP = _B
_LB = (-(-_B // _LCAP)) * _LCAP

_VOCABS = {"host": 1000000, "dom": 100000, "cat": 1000}


def _shard(v):
    per_w = -(-v // _NW)
    return 128 * (-(-per_w // 128))


def _gather_table(tab_hbm, idx_hbm, out_hbm, v, wid, idx_v,
                  cposp_v, seg_bufs, row32_v, rb_v, sems, ssem):
    shard = _shard(v)
    nseg = -(-shard // _S)
    if nseg % 2:
        nseg += 1
    lastbase = ((v - _S) // 128) * 128
    lo = wid * shard
    hi = lo + shard

    pltpu.sync_copy(idx_hbm, idx_v.at[pl.ds(0, _B)])

    def compact(k4, cnt):
        for u in range(4):
            k = k4 * 4 + u
            iv = idx_v[pl.ds(k * _L, _L)]
            m = (iv >= lo) & (iv < hi)
            dst = cnt + plsc.cumsum(m.astype(jnp.int32)) - 1
            pos = k * _L + lax.iota(jnp.int32, _L)
            entry = ((iv - lo) << 15) | pos
            plsc.store_scatter(idx_v, [dst], entry, mask=m)
            cnt = cnt + jnp.sum(m.astype(jnp.int32), axis=0)
        return cnt
    cnt = lax.fori_loop(0, _B // _L // 4, compact, jnp.int32(0))

    npass = (cnt + (_LCAP - 1)) // _LCAP

    sentinel = (0x7FFF << 15) | _DUMP

    def filltail(t, _):
        j = cnt + t * _L
        plsc.store_scatter(
            idx_v, [j + lax.iota(jnp.int32, _L)],
            jnp.full((_L,), sentinel, jnp.int32),
            mask=(j + lax.iota(jnp.int32, _L)) < _LB)
        return 0
    lax.fori_loop(0, (npass * _LCAP - cnt + _L - 1) // _L, filltail, 0)

    def wbase(s):
        return pl.multiple_of(jnp.minimum(lo + s * _S, lastbase), 128)

    def stage(s, par):
        return pltpu.async_copy(
            tab_hbm.at[:, pl.ds(wbase(s), _S)], seg_bufs[par], sems[par])

    def scan_window(p, base, buf):
        lbase = base - lo

        def vreg(r4, _):
            for u in range(4):
                r = r4 * 4 + u
                g = p * _LCAP + r * _L
                ev = idx_v[pl.ds(g, _L)]
                liv = ev >> 15
                m = (liv >= lbase) & (liv < lbase + _S)

                @pl.when(jnp.any(m))
                def _(liv=liv, m=m, r=r):
                    lv = liv - lbase
                    rows = r * _L + lax.iota(jnp.int32, _L)
                    for c in range(_EMBED_DIM):
                        cc = jnp.full((_L,), c, jnp.int32)
                        vals = plsc.load_gather(buf, [cc, lv], mask=m)
                        plsc.store_scatter(row32_v, [rows, cc], vals, mask=m)
            return 0
        lax.fori_loop(0, _LCAP // _L // 4, vreg, 0)

    def one_pass(p, _):
        stage(0, 0)
        stage(1, 1)

        def segpair(sh, _):
            for par in range(2):
                s = sh * 2 + par
                base = wbase(s)
                pltpu.make_async_copy(
                    tab_hbm.at[:, pl.ds(base, _S)], seg_bufs[par],
                    sems[par]).wait()
                scan_window(p, base, seg_bufs[par])

                @pl.when(s + 2 < nseg)
                def _():
                    stage(s + 2, par)
            return 0
        lax.fori_loop(0, nseg // 2, segpair, 0)

        handle = None
        for q in range(_LCAP // _SCH):
            if handle is not None:
                handle.wait()

            def expand(rr, _, _q=q):
                src = _q * _SCH + rr
                rb_v[rr, pl.ds(0, _L)] = row32_v[src, pl.ds(0, _L)]
                rb_v[rr, pl.ds(_L, _L)] = row32_v[src, pl.ds(_L, _L)]
                return 0
            lax.fori_loop(0, _SCH, expand, 0)

            def poscopy(j, _, _q=q):
                g = p * _LCAP + _q * _SCH + j * _L
                ev = idx_v[pl.ds(g, _L)]
                cposp_v[pl.ds(j * _L, _L)] = ev & ((1 << 15) - 1)
                return 0
            lax.fori_loop(0, _SCH // _L, poscopy, 0)
            handle = pltpu.async_copy(rb_v, out_hbm.at[cposp_v], ssem)
        handle.wait()
        return 0

    lax.fori_loop(0, npass, one_pass, 0)


@functools.cache
def _make_sc_gather3():
    mesh = plsc.VectorSubcoreMesh(core_axis_name="c", subcore_axis_name="s")

    @functools.partial(
        pl.kernel,
        out_type=tuple(
            jax.ShapeDtypeStruct((_B + 8, _OUTPUT_DIM), jnp.float32)
            for _ in range(3)),
        mesh=mesh,
        scratch_types=[
            pltpu.VMEM((_LB,), jnp.int32),
            pltpu.VMEM((_SCH,), jnp.int32),
            pltpu.VMEM((_EMBED_DIM, _S), jnp.float32),
            pltpu.VMEM((_EMBED_DIM, _S), jnp.float32),
            pltpu.VMEM((_LCAP, _EMBED_DIM), jnp.float32),
            pltpu.VMEM((_SCH, _OUTPUT_DIM), jnp.float32),
            pltpu.SemaphoreType.DMA,
            pltpu.SemaphoreType.DMA,
            pltpu.SemaphoreType.DMA,
        ],
        compiler_params=pltpu.CompilerParams(needs_layout_passes=False),
    )
    def _sc_gather3(cat_idx_hbm, host_idx_hbm, dom_idx_hbm,
                    cat_t_hbm, host_t_hbm, dom_t_hbm,
                    cat_out, host_out, dom_out,
                    idx_v, cposp_v, seg0_v, seg1_v, row32_v, rb_v,
                    s0, s1, s2):
        wid = lax.axis_index("s") * _NC + lax.axis_index("c")
        for idx_hbm, tab_hbm, out_hbm, v in (
                (host_idx_hbm, host_t_hbm, host_out, _VOCABS["host"]),
                (dom_idx_hbm, dom_t_hbm, dom_out, _VOCABS["dom"]),
                (cat_idx_hbm, cat_t_hbm, cat_out, _VOCABS["cat"]),
        ):
            _gather_table(tab_hbm, idx_hbm, out_hbm, v, wid,
                          idx_v, cposp_v, (seg0_v, seg1_v), row32_v, rb_v,
                          (s0, s1), s2)

    return _sc_gather3


_ROWS = 2048


def _dot0(a, w):
    return lax.dot_general(a, w, (((0,), (0,)), ((), ())),
                           preferred_element_type=jnp.float32)


def _emb_h(g_ref, idx_ref, m_ref, w_slice, v):
    tail_base = v - (v % 128)
    idx = idx_ref[...]
    tm = idx >= tail_base
    emb = jnp.where(tm, 0.0, g_ref[...][:, :_EMBED_DIM])
    h = jnp.dot(emb, w_slice, preferred_element_type=jnp.float32)
    if v % 128:
        rows = idx.shape[0]
        lane = lax.broadcasted_iota(jnp.int32, (rows, _OUTPUT_DIM), 1)
        oh = ((idx - tail_base) == lane) & tm
        h = h + jnp.dot(oh.astype(jnp.float32), m_ref[...],
                        preferred_element_type=jnp.float32)
    return h


def _tc_body(num_ref, ci_ref, hi_ref, di_ref, cat_ref, host_ref, dom_ref,
             w_ref, mc_ref, mh_ref, md_ref, b_ref, g_ref, be_ref, out_ref):
    w = w_ref[...]
    h = (_dot0(num_ref[...], w[0:16])
         + _emb_h(cat_ref, ci_ref, mc_ref, w[16:48], _VOCABS["cat"])
         + _emb_h(host_ref, hi_ref, mh_ref, w[48:80], _VOCABS["host"])
         + _emb_h(dom_ref, di_ref, md_ref, w[80:112], _VOCABS["dom"])
         + b_ref[...])
    mean = jnp.mean(h, axis=-1, keepdims=True)
    var = jnp.mean(jnp.square(h - mean), axis=-1, keepdims=True)
    y = (h - mean) * lax.rsqrt(var + 1e-5) * g_ref[...] + be_ref[...]
    out_ref[...] = y * 0.5 * (1.0 + lax.erf(y * 0.7071067811865476))


def _tc_dense(num_t, ci2, hi2, di2, cat_g, host_g, dom_g,
              W, mc, mh, md, b, gamma, beta):
    grid = _B // _ROWS
    full = lambda i: (0, 0)
    return pl.pallas_call(
        _tc_body,
        grid=(grid,),
        in_specs=[
            pl.BlockSpec((_NUMERIC_DIM, _ROWS), lambda i: (0, i)),
            pl.BlockSpec((_ROWS, 1), lambda i: (i, 0)),
            pl.BlockSpec((_ROWS, 1), lambda i: (i, 0)),
            pl.BlockSpec((_ROWS, 1), lambda i: (i, 0)),
            pl.BlockSpec((_ROWS, _OUTPUT_DIM), lambda i: (i, 0)),
            pl.BlockSpec((_ROWS, _OUTPUT_DIM), lambda i: (i, 0)),
            pl.BlockSpec((_ROWS, _OUTPUT_DIM), lambda i: (i, 0)),
            pl.BlockSpec((_NUMERIC_DIM + 3 * _EMBED_DIM, _OUTPUT_DIM), full),
            pl.BlockSpec((_OUTPUT_DIM, _OUTPUT_DIM), full),
            pl.BlockSpec((_OUTPUT_DIM, _OUTPUT_DIM), full),
            pl.BlockSpec((_OUTPUT_DIM, _OUTPUT_DIM), full),
            pl.BlockSpec((1, _OUTPUT_DIM), full),
            pl.BlockSpec((1, _OUTPUT_DIM), full),
            pl.BlockSpec((1, _OUTPUT_DIM), full),
        ],
        out_specs=pl.BlockSpec((_ROWS, _OUTPUT_DIM), lambda i: (i, 0)),
        out_shape=jax.ShapeDtypeStruct((_B, _OUTPUT_DIM), jnp.float32),
    )(num_t, ci2, hi2, di2, cat_g, host_g, dom_g, W, mc, mh, md,
      b.reshape(1, _OUTPUT_DIM), gamma.reshape(1, _OUTPUT_DIM),
      beta.reshape(1, _OUTPUT_DIM))


def _tail_matrix(table, w_slice):
    v = table.shape[0]
    tw = v % 128
    tail = table[v - tw:]
    return jnp.pad(tail, ((0, 128 - tw), (0, 0))) @ w_slice


def kernel(meta_numeric, meta_category_id, meta_host_id, meta_domain_id,
           cat_table, host_table, domain_table, W, b, gamma, beta):
    ci = meta_category_id.astype(jnp.int32)
    hi = meta_host_id.astype(jnp.int32)
    di = meta_domain_id.astype(jnp.int32)
    cat_g, host_g, dom_g = _make_sc_gather3()(
        ci, hi, di, cat_table.T, host_table.T, domain_table.T)
    mc = _tail_matrix(cat_table, W[16:48])
    mh = _tail_matrix(host_table, W[48:80])
    md = _tail_matrix(domain_table, W[80:112])
    return _tc_dense(meta_numeric.T, ci.reshape(-1, 1), hi.reshape(-1, 1),
                     di.reshape(-1, 1), cat_g, host_g, dom_g,
                     W, mc, mh, md, b, gamma, beta)
